# baseline (speedup 1.0000x reference)
import jax
import jax.numpy as jnp
from jax.experimental import pallas as pl
from jax.experimental.pallas import tpu as pltpu

_CHUNKS = [4000, 12000, 12000, 12000, 12000, 12000, 12000, 12000, 12000]
_MAXC = max(_CHUNKS)
_NBUF = 4
_OFFS = [sum(_CHUNKS[:i]) for i in range(len(_CHUNKS))]


def _mm_bias_kernel(x_hbm, w_ref, b_ref, o_hbm, x_buf, o_buf, in_sems, out_sems):
    nchunk = len(_CHUNKS)

    def in_copy(i, s):
        c = _CHUNKS[i]
        return pltpu.make_async_copy(
            x_hbm.at[pl.ds(_OFFS[i], c), :],
            x_buf.at[s, pl.ds(0, c), :],
            in_sems.at[s],
        )

    def out_copy(i, s):
        c = _CHUNKS[i]
        return pltpu.make_async_copy(
            o_buf.at[s, pl.ds(0, c), :],
            o_hbm.at[pl.ds(_OFFS[i], c), :],
            out_sems.at[s],
        )

    for s in range(_NBUF):
        in_copy(s, s).start()

    w = w_ref[...]
    b = b_ref[...]

    for i in range(nchunk):
        s = i % _NBUF
        c = _CHUNKS[i]
        in_copy(i, s).wait()
        o = (
            jnp.dot(
                x_buf[s, pl.ds(0, c), :], w, preferred_element_type=jnp.float32
            )
            + b
        )
        if i >= _NBUF:
            out_copy(i - _NBUF, s).wait()
        o_buf[s, pl.ds(0, c), :] = o
        out_copy(i, s).start()
        if i + _NBUF < nchunk:
            in_copy(i + _NBUF, s).start()

    for i in range(nchunk - _NBUF, nchunk):
        out_copy(i, i % _NBUF).wait()


def kernel(input, kernel, bias):
    n, cin = input.shape
    cout = kernel.shape[1]
    return pl.pallas_call(
        _mm_bias_kernel,
        in_specs=[
            pl.BlockSpec(memory_space=pltpu.MemorySpace.HBM),
            pl.BlockSpec((cin, cout), lambda: (0, 0)),
            pl.BlockSpec((1, cout), lambda: (0, 0)),
        ],
        out_specs=pl.BlockSpec(memory_space=pltpu.MemorySpace.HBM),
        out_shape=jax.ShapeDtypeStruct((n, cout), jnp.float32),
        scratch_shapes=[
            pltpu.VMEM((_NBUF, _MAXC, cin), jnp.float32),
            pltpu.VMEM((_NBUF, _MAXC, cout), jnp.float32),
            pltpu.SemaphoreType.DMA((_NBUF,)),
            pltpu.SemaphoreType.DMA((_NBUF,)),
        ],
    )(input, kernel, bias)


# speedup vs baseline: 1.0388x; 1.0388x over previous
import jax
import jax.numpy as jnp
from jax.experimental import pallas as pl
from jax.experimental.pallas import tpu as pltpu

_CHUNKS = [8000, 16000, 16000, 16000, 16000, 16000, 8000, 4000]
_MAXC = max(_CHUNKS)
_NBUF = 3
_OFFS = [sum(_CHUNKS[:i]) for i in range(len(_CHUNKS))]


def _mm_bias_kernel(x_hbm, w_ref, b_ref, o_hbm, x_buf, o_buf, in_sems, out_sems):
    nchunk = len(_CHUNKS)

    def in_copy(i, s):
        c = _CHUNKS[i]
        return pltpu.make_async_copy(
            x_hbm.at[pl.ds(_OFFS[i], c), :],
            x_buf.at[s, pl.ds(0, c), :],
            in_sems.at[s],
        )

    def out_copy(i, s):
        c = _CHUNKS[i]
        return pltpu.make_async_copy(
            o_buf.at[s, pl.ds(0, c), :],
            o_hbm.at[pl.ds(_OFFS[i], c), :],
            out_sems.at[s],
        )

    for s in range(_NBUF):
        in_copy(s, s).start()

    w = w_ref[...]
    b = b_ref[...]

    for i in range(nchunk):
        s = i % _NBUF
        c = _CHUNKS[i]
        in_copy(i, s).wait()
        o = (
            jnp.dot(
                x_buf[s, pl.ds(0, c), :], w, preferred_element_type=jnp.float32
            )
            + b
        )
        if i >= _NBUF:
            out_copy(i - _NBUF, s).wait()
        o_buf[s, pl.ds(0, c), :] = o
        out_copy(i, s).start()
        if i + _NBUF < nchunk:
            in_copy(i + _NBUF, s).start()

    for i in range(nchunk - _NBUF, nchunk):
        out_copy(i, i % _NBUF).wait()


def kernel(input, kernel, bias):
    n, cin = input.shape
    cout = kernel.shape[1]
    return pl.pallas_call(
        _mm_bias_kernel,
        in_specs=[
            pl.BlockSpec(memory_space=pltpu.MemorySpace.HBM),
            pl.BlockSpec((cin, cout), lambda: (0, 0)),
            pl.BlockSpec((1, cout), lambda: (0, 0)),
        ],
        out_specs=pl.BlockSpec(memory_space=pltpu.MemorySpace.HBM),
        out_shape=jax.ShapeDtypeStruct((n, cout), jnp.float32),
        scratch_shapes=[
            pltpu.VMEM((_NBUF, _MAXC, cin), jnp.float32),
            pltpu.VMEM((_NBUF, _MAXC, cout), jnp.float32),
            pltpu.SemaphoreType.DMA((_NBUF,)),
            pltpu.SemaphoreType.DMA((_NBUF,)),
        ],
    )(input, kernel, bias)


# speedup vs baseline: 1.0452x; 1.0061x over previous
import jax
import jax.numpy as jnp
from jax.experimental import pallas as pl
from jax.experimental.pallas import tpu as pltpu

_CHUNKS = [4000, 16000, 16000, 16000, 16000, 16000, 12000, 4000]
_MAXC = max(_CHUNKS)
_NBUF = 3
_OFFS = [sum(_CHUNKS[:i]) for i in range(len(_CHUNKS))]


def _mm_bias_kernel(x_hbm, w_ref, b_ref, o_hbm, x_buf, o_buf, in_sems, out_sems):
    nchunk = len(_CHUNKS)

    def in_copy(i, s):
        c = _CHUNKS[i]
        return pltpu.make_async_copy(
            x_hbm.at[pl.ds(_OFFS[i], c), :],
            x_buf.at[s, pl.ds(0, c), :],
            in_sems.at[s],
        )

    def out_copy(i, s):
        c = _CHUNKS[i]
        return pltpu.make_async_copy(
            o_buf.at[s, pl.ds(0, c), :],
            o_hbm.at[pl.ds(_OFFS[i], c), :],
            out_sems.at[s],
        )

    for s in range(_NBUF):
        in_copy(s, s).start()

    w = w_ref[...]
    b = b_ref[...]

    for i in range(nchunk):
        s = i % _NBUF
        c = _CHUNKS[i]
        in_copy(i, s).wait()
        o = (
            jnp.dot(
                x_buf[s, pl.ds(0, c), :], w, preferred_element_type=jnp.float32
            )
            + b
        )
        if i >= _NBUF:
            out_copy(i - _NBUF, s).wait()
        o_buf[s, pl.ds(0, c), :] = o
        out_copy(i, s).start()
        if i + _NBUF < nchunk:
            in_copy(i + _NBUF, s).start()

    for i in range(nchunk - _NBUF, nchunk):
        out_copy(i, i % _NBUF).wait()


def kernel(input, kernel, bias):
    n, cin = input.shape
    cout = kernel.shape[1]
    return pl.pallas_call(
        _mm_bias_kernel,
        in_specs=[
            pl.BlockSpec(memory_space=pltpu.MemorySpace.HBM),
            pl.BlockSpec((cin, cout), lambda: (0, 0)),
            pl.BlockSpec((1, cout), lambda: (0, 0)),
        ],
        out_specs=pl.BlockSpec(memory_space=pltpu.MemorySpace.HBM),
        out_shape=jax.ShapeDtypeStruct((n, cout), jnp.float32),
        scratch_shapes=[
            pltpu.VMEM((_NBUF, _MAXC, cin), jnp.float32),
            pltpu.VMEM((_NBUF, _MAXC, cout), jnp.float32),
            pltpu.SemaphoreType.DMA((_NBUF,)),
            pltpu.SemaphoreType.DMA((_NBUF,)),
        ],
    )(input, kernel, bias)
